# Initial kernel scaffold; baseline (speedup 1.0000x reference)
import functools

import jax
import jax.numpy as jnp
from jax import lax
from jax.experimental import pallas as pl
from jax.experimental.pallas import tpu as pltpu
from jax.experimental.pallas import tpu_sc as plsc

N_CAND = 8
B, S, D = 2, 2048, 1024
SLAB_ROWS = B * S
SLAB_ELEMS = SLAB_ROWS * D
TOTAL_ELEMS = N_CAND * SLAB_ELEMS

NW = 32
ROWS_PER_W = SLAB_ROWS // NW
CHUNK_ROWS = 16
CHUNK_ELEMS = CHUNK_ROWS * D
N_CHUNKS = ROWS_PER_W // CHUNK_ROWS
LANES = 16

_mesh = plsc.VectorSubcoreMesh(core_axis_name="c", subcore_axis_name="s")


@functools.partial(
    pl.kernel,
    out_type=jax.ShapeDtypeStruct((SLAB_ELEMS,), jnp.float32),
    mesh=_mesh,
    scratch_types=[
        pltpu.VMEM((LANES,), jnp.int32),
        pltpu.VMEM((CHUNK_ELEMS,), jnp.float32),
        pltpu.VMEM((CHUNK_ELEMS,), jnp.float32),
        pltpu.VMEM((CHUNK_ELEMS,), jnp.float32),
        pltpu.VMEM((CHUNK_ELEMS,), jnp.float32),
        pltpu.SemaphoreType.DMA,
        pltpu.SemaphoreType.DMA,
        pltpu.SemaphoreType.DMA,
        pltpu.SemaphoreType.DMA,
        pltpu.SemaphoreType.DMA,
        pltpu.SemaphoreType.DMA,
    ],
)
def _avg_pair(in_ref, dsel_ref, out_ref, dvec, a0, b0, a1, b1,
              sa0, sb0, sa1, sb1, so0, so1):
    cid = lax.axis_index("c")
    sid = lax.axis_index("s")
    wid = sid * 2 + cid

    pltpu.sync_copy(dsel_ref, dvec)
    d = jnp.max(dvec[...])
    abase = d * (2 * SLAB_ELEMS) + wid * (ROWS_PER_W * D)
    bbase = abase + SLAB_ELEMS
    obase = wid * (ROWS_PER_W * D)

    bufs = ((a0, b0, sa0, sb0, so0), (a1, b1, sa1, sb1, so1))

    def start_in(g):
        a, b, sa, sb, _ = bufs[g % 2]
        off = g * CHUNK_ELEMS
        da = pltpu.async_copy(in_ref.at[pl.ds(abase + off, CHUNK_ELEMS)], a, sa)
        db = pltpu.async_copy(in_ref.at[pl.ds(bbase + off, CHUNK_ELEMS)], b, sb)
        return da, db

    out_dmas = [None, None]
    pend = start_in(0)
    for g in range(N_CHUNKS):
        a, b, _, _, so = bufs[g % 2]
        nxt = None
        if g + 1 < N_CHUNKS:
            if out_dmas[(g + 1) % 2] is not None:
                out_dmas[(g + 1) % 2].wait()
                out_dmas[(g + 1) % 2] = None
            nxt = start_in(g + 1)
        pend[0].wait()
        pend[1].wait()

        half = jnp.float32(0.5)

        @plsc.parallel_loop(0, CHUNK_ELEMS, step=LANES, unroll=8)
        def _(i):
            a[pl.ds(i, LANES)] = (a[pl.ds(i, LANES)] + b[pl.ds(i, LANES)]) * half

        out_dmas[g % 2] = pltpu.async_copy(
            a, out_ref.at[pl.ds(obase + g * CHUNK_ELEMS, CHUNK_ELEMS)], so)
        pend = nxt

    for od in out_dmas:
        if od is not None:
            od.wait()


def kernel(inputs, domain_idx):
    flat = inputs.reshape(TOTAL_ELEMS)
    dsel = jnp.full((LANES,), jnp.asarray(domain_idx, jnp.int32), jnp.int32)
    out = _avg_pair(flat, dsel)
    return out.reshape(B, S, D)


# speedup vs baseline: 1.1964x; 1.1964x over previous
import functools

import jax
import jax.numpy as jnp
from jax import lax
from jax.experimental import pallas as pl
from jax.experimental.pallas import tpu as pltpu
from jax.experimental.pallas import tpu_sc as plsc

N_CAND = 8
B, S, D = 2, 2048, 1024
SLAB_ROWS = B * S
SLAB_ELEMS = SLAB_ROWS * D
TOTAL_ELEMS = N_CAND * SLAB_ELEMS

NW = 32
ROWS_PER_W = SLAB_ROWS // NW
CHUNK_ROWS = 16
CHUNK_ELEMS = CHUNK_ROWS * D
N_CHUNKS = ROWS_PER_W // CHUNK_ROWS
LANES = 16

def _avg_pair_impl(in_ref, dsel_ref, out_ref, dvec, a0, b0, a1, b1,
                   sa0, sb0, sa1, sb1, so0, so1):
    cid = lax.axis_index("c")
    sid = lax.axis_index("s")
    wid = sid * 2 + cid

    pltpu.sync_copy(dsel_ref, dvec)
    d = dvec[...][0]
    abase = d * (2 * SLAB_ELEMS) + wid * (ROWS_PER_W * D)
    bbase = abase + SLAB_ELEMS
    obase = wid * (ROWS_PER_W * D)

    bufs = ((a0, b0, sa0, sb0, so0), (a1, b1, sa1, sb1, so1))

    def start_in(g):
        a, b, sa, sb, _ = bufs[g % 2]
        off = g * CHUNK_ELEMS
        da = pltpu.async_copy(in_ref.at[pl.ds(abase + off, CHUNK_ELEMS)], a, sa)
        db = pltpu.async_copy(in_ref.at[pl.ds(bbase + off, CHUNK_ELEMS)], b, sb)
        return da, db

    out_dmas = [None, None]
    pend = start_in(0)
    for g in range(N_CHUNKS):
        a, b, _, _, so = bufs[g % 2]
        nxt = None
        if g + 1 < N_CHUNKS:
            if out_dmas[(g + 1) % 2] is not None:
                out_dmas[(g + 1) % 2].wait()
                out_dmas[(g + 1) % 2] = None
            nxt = start_in(g + 1)
        pend[0].wait()
        pend[1].wait()

        half = jnp.float32(0.5)

        @plsc.parallel_loop(0, CHUNK_ELEMS, step=LANES, unroll=8)
        def _(i):
            a[pl.ds(i, LANES)] = (a[pl.ds(i, LANES)] + b[pl.ds(i, LANES)]) * half

        out_dmas[g % 2] = pltpu.async_copy(
            a, out_ref.at[pl.ds(obase + g * CHUNK_ELEMS, CHUNK_ELEMS)], so)
        pend = nxt

    for od in out_dmas:
        if od is not None:
            od.wait()


@functools.lru_cache(maxsize=1)
def _build_avg_pair():
    mesh = plsc.VectorSubcoreMesh(core_axis_name="c", subcore_axis_name="s")
    return pl.kernel(
        _avg_pair_impl,
        out_type=jax.ShapeDtypeStruct((SLAB_ELEMS,), jnp.float32),
        mesh=mesh,
        scratch_types=[
            pltpu.VMEM((LANES,), jnp.int32),
            pltpu.VMEM((CHUNK_ELEMS,), jnp.float32),
            pltpu.VMEM((CHUNK_ELEMS,), jnp.float32),
            pltpu.VMEM((CHUNK_ELEMS,), jnp.float32),
            pltpu.VMEM((CHUNK_ELEMS,), jnp.float32),
            pltpu.SemaphoreType.DMA,
            pltpu.SemaphoreType.DMA,
            pltpu.SemaphoreType.DMA,
            pltpu.SemaphoreType.DMA,
            pltpu.SemaphoreType.DMA,
            pltpu.SemaphoreType.DMA,
        ],
    )


def kernel(inputs, domain_idx):
    flat = inputs.reshape(TOTAL_ELEMS)
    dsel = jnp.full((LANES,), jnp.asarray(domain_idx, jnp.int32), jnp.int32)
    out = _build_avg_pair()(flat, dsel)
    return out.reshape(B, S, D)


# speedup vs baseline: 4.4699x; 3.7363x over previous
import functools

import jax
import jax.numpy as jnp
from jax import lax
from jax.experimental import pallas as pl
from jax.experimental.pallas import tpu as pltpu
from jax.experimental.pallas import tpu_sc as plsc

N_CAND = 8
B, S, D = 2, 2048, 1024
SLAB_ROWS = B * S
TOTAL_ROWS = N_CAND * SLAB_ROWS

NW = 32
ROWS_PER_W = SLAB_ROWS // NW
CHUNK_ROWS = 16
N_CHUNKS = ROWS_PER_W // CHUNK_ROWS
LANES = 16
COL_GROUPS = D // LANES


def _avg_pair_impl(in_ref, dsel_ref, out_ref, dvec, a0, b0, a1, b1,
                   sa0, sb0, sa1, sb1, so0, so1):
    cid = lax.axis_index("c")
    sid = lax.axis_index("s")
    wid = sid * 2 + cid

    pltpu.sync_copy(dsel_ref, dvec)
    d = dvec[...][0]
    arow = d * (2 * SLAB_ROWS) + wid * ROWS_PER_W
    brow = arow + SLAB_ROWS
    orow = wid * ROWS_PER_W

    bufs = ((a0, b0, sa0, sb0, so0), (a1, b1, sa1, sb1, so1))

    def start_in(g):
        a, b, sa, sb, _ = bufs[g % 2]
        off = g * CHUNK_ROWS
        da = pltpu.async_copy(in_ref.at[pl.ds(arow + off, CHUNK_ROWS)], a, sa)
        db = pltpu.async_copy(in_ref.at[pl.ds(brow + off, CHUNK_ROWS)], b, sb)
        return da, db

    half = jnp.float32(0.5)
    out_dmas = [None, None]
    pend = start_in(0)
    for g in range(N_CHUNKS):
        a, b, _, _, so = bufs[g % 2]
        nxt = None
        if g + 1 < N_CHUNKS:
            if out_dmas[(g + 1) % 2] is not None:
                out_dmas[(g + 1) % 2].wait()
                out_dmas[(g + 1) % 2] = None
            nxt = start_in(g + 1)
        pend[0].wait()
        pend[1].wait()

        @plsc.parallel_loop(0, CHUNK_ROWS * COL_GROUPS, step=1, unroll=8)
        def _(i):
            r = i >> 6
            c = (i & (COL_GROUPS - 1)) * LANES
            a[r, pl.ds(c, LANES)] = (
                a[r, pl.ds(c, LANES)] + b[r, pl.ds(c, LANES)]) * half

        out_dmas[g % 2] = pltpu.async_copy(
            a, out_ref.at[pl.ds(orow + g * CHUNK_ROWS, CHUNK_ROWS)], so)
        pend = nxt

    for od in out_dmas:
        if od is not None:
            od.wait()


@functools.lru_cache(maxsize=1)
def _build_avg_pair():
    mesh = plsc.VectorSubcoreMesh(core_axis_name="c", subcore_axis_name="s")
    return pl.kernel(
        _avg_pair_impl,
        out_type=jax.ShapeDtypeStruct((SLAB_ROWS, D), jnp.float32),
        mesh=mesh,
        compiler_params=pltpu.CompilerParams(use_tc_tiling_on_sc=True),
        scratch_types=[
            pltpu.VMEM((LANES,), jnp.int32),
            pltpu.VMEM((CHUNK_ROWS, D), jnp.float32),
            pltpu.VMEM((CHUNK_ROWS, D), jnp.float32),
            pltpu.VMEM((CHUNK_ROWS, D), jnp.float32),
            pltpu.VMEM((CHUNK_ROWS, D), jnp.float32),
            pltpu.SemaphoreType.DMA,
            pltpu.SemaphoreType.DMA,
            pltpu.SemaphoreType.DMA,
            pltpu.SemaphoreType.DMA,
            pltpu.SemaphoreType.DMA,
            pltpu.SemaphoreType.DMA,
        ],
    )


def kernel(inputs, domain_idx):
    rows = inputs.reshape(TOTAL_ROWS, D)
    dsel = jnp.full((LANES,), jnp.asarray(domain_idx, jnp.int32), jnp.int32)
    out = _build_avg_pair()(rows, dsel)
    return out.reshape(B, S, D)
